# baseline (speedup 1.0000x reference)
import functools

import jax
import jax.numpy as jnp
from jax import lax
from jax.experimental import pallas as pl
from jax.experimental.pallas import tpu as pltpu
from jax.experimental.pallas import tpu_sc as plsc

D_MODEL = 128
BATCH = 16384

_info = plsc.get_sparse_core_info()
NC, NS = _info.num_cores, _info.num_subcores
NW = NC * NS
B_PER_W = BATCH // NW
CHUNK = 128
NCH = B_PER_W // CHUNK

_mesh = plsc.VectorSubcoreMesh(core_axis_name="c", subcore_axis_name="s")


@functools.partial(
    pl.kernel,
    mesh=_mesh,
    out_type=[
        jax.ShapeDtypeStruct((BATCH, D_MODEL), jnp.float32),
        jax.ShapeDtypeStruct((BATCH,), jnp.float32),
    ],
    scratch_types=[
        pltpu.VMEM((NCH, CHUNK), jnp.int32),
        pltpu.VMEM((B_PER_W, D_MODEL), jnp.float32),
        pltpu.VMEM_SHARED((NS * 2 * CHUNK, D_MODEL), jnp.float32),
        pltpu.VMEM((B_PER_W,), jnp.float32),
    ]
    + [pltpu.SemaphoreType.DMA for _ in range(NCH)]
    + [pltpu.SemaphoreType.DMA for _ in range(2)]
    + [pltpu.SemaphoreType.DMA for _ in range(2)]
    + [pltpu.SemaphoreType.DMA, pltpu.SemaphoreType.DMA],
)
def _gather_kernel(idx_hbm, weight_hbm, bias_hbm, w_out, b_out,
                   idx_v, rows_v, rows_sh, bias_v, *sems):
    gsems = sems[:NCH]
    xsems = sems[NCH:NCH + 2]
    dsems = sems[NCH + 2:NCH + 4]
    bsem, ssem = sems[NCH + 4], sems[NCH + 5]
    cid = lax.axis_index("c")
    sid = lax.axis_index("s")
    wid = sid * NC + cid
    base = wid * B_PER_W
    sbase = sid * B_PER_W
    pltpu.sync_copy(idx_hbm.at[wid], idx_v)
    bcopies = []
    for j in range(NCH):
        bcopies.append(pltpu.async_copy(
            bias_hbm.at[idx_v.at[j]],
            bias_v.at[pl.ds(j * CHUNK, CHUNK)], bsem))
    wcopies = []
    for j in range(NCH):
        wcopies.append(pltpu.async_copy(
            weight_hbm.at[idx_v.at[j]],
            rows_v.at[pl.ds(j * CHUNK, CHUNK)], gsems[j]))
    drains = [None, None]
    for j in range(NCH):
        slot = j % 2
        soff = (sid * 2 + slot) * CHUNK
        wcopies[j].wait()
        if drains[slot] is not None:
            drains[slot].wait()
        pltpu.async_copy(
            rows_v.at[pl.ds(j * CHUNK, CHUNK)],
            rows_sh.at[pl.ds(soff, CHUNK)], xsems[slot]).wait()
        drains[slot] = pltpu.async_copy(
            rows_sh.at[pl.ds(soff, CHUNK)],
            w_out.at[pl.ds(base + j * CHUNK, CHUNK)], dsems[slot])
    for c in bcopies:
        c.wait()
    bstore = pltpu.async_copy(bias_v, b_out.at[pl.ds(base, B_PER_W)], ssem)
    for d in drains:
        if d is not None:
            d.wait()
    bstore.wait()


def kernel(coords, voxel_indices, weight, bias):
    del coords
    idx = voxel_indices.astype(jnp.int32).reshape(NW, NCH, CHUNK)
    w, b = _gather_kernel(idx, weight, bias)
    return (w, b)


# speedup vs baseline: 1.0760x; 1.0760x over previous
import functools

import jax
import jax.numpy as jnp
from jax import lax
from jax.experimental import pallas as pl
from jax.experimental.pallas import tpu as pltpu
from jax.experimental.pallas import tpu_sc as plsc

D_MODEL = 128
BATCH = 16384

_info = plsc.get_sparse_core_info()
NC, NS = _info.num_cores, _info.num_subcores
NW = NC * NS
B_PER_W = BATCH // NW
CHUNK = 128
NCH = B_PER_W // CHUNK

_mesh = plsc.VectorSubcoreMesh(core_axis_name="c", subcore_axis_name="s")


@functools.partial(
    pl.kernel,
    mesh=_mesh,
    out_type=[
        jax.ShapeDtypeStruct((BATCH, D_MODEL), jnp.float32),
        jax.ShapeDtypeStruct((BATCH,), jnp.float32),
    ],
    scratch_types=[
        pltpu.VMEM((NCH, CHUNK), jnp.int32),
        pltpu.VMEM((B_PER_W, D_MODEL), jnp.float32),
        pltpu.VMEM((B_PER_W,), jnp.float32),
    ]
    + [pltpu.SemaphoreType.DMA for _ in range(NCH)]
    + [pltpu.SemaphoreType.DMA, pltpu.SemaphoreType.DMA],
)
def _gather_kernel(idx_hbm, weight_hbm, bias_hbm, w_out, b_out,
                   idx_v, rows_v, bias_v, *sems):
    gsems, bsem, ssem = sems[:NCH], sems[NCH], sems[NCH + 1]
    wid = lax.axis_index("s") * NC + lax.axis_index("c")
    base = wid * B_PER_W
    icopies = [pltpu.async_copy(idx_hbm.at[wid, j], idx_v.at[j], ssem)
               for j in range(NCH)]
    wcopies, bcopies = [], []
    for j in range(NCH):
        icopies[j].wait()
        wcopies.append(pltpu.async_copy(
            weight_hbm.at[idx_v.at[j]],
            rows_v.at[pl.ds(j * CHUNK, CHUNK)], gsems[j]))
        bcopies.append(pltpu.async_copy(
            bias_hbm.at[idx_v.at[j]],
            bias_v.at[pl.ds(j * CHUNK, CHUNK)], bsem))
    stores = []
    for j in range(NCH):
        wcopies[j].wait()
        stores.append(pltpu.async_copy(
            rows_v.at[pl.ds(j * CHUNK, CHUNK)],
            w_out.at[pl.ds(base + j * CHUNK, CHUNK)], ssem))
    for c in bcopies:
        c.wait()
    stores.append(pltpu.async_copy(bias_v, b_out.at[pl.ds(base, B_PER_W)], ssem))
    for s in stores:
        s.wait()


def kernel(coords, voxel_indices, weight, bias):
    del coords
    idx = voxel_indices.astype(jnp.int32).reshape(NW, NCH, CHUNK)
    w, b = _gather_kernel(idx, weight, bias)
    return (w, b)


# speedup vs baseline: 1.0842x; 1.0077x over previous
import functools

import jax
import jax.numpy as jnp
from jax import lax
from jax.experimental import pallas as pl
from jax.experimental.pallas import tpu as pltpu
from jax.experimental.pallas import tpu_sc as plsc

D_MODEL = 128
BATCH = 16384

_info = plsc.get_sparse_core_info()
NC, NS = _info.num_cores, _info.num_subcores
NW = NC * NS
B_PER_W = BATCH // NW
CHUNK = 128
NCH = B_PER_W // CHUNK

_mesh = plsc.VectorSubcoreMesh(core_axis_name="c", subcore_axis_name="s")


@functools.partial(
    pl.kernel,
    mesh=_mesh,
    out_type=[
        jax.ShapeDtypeStruct((BATCH, D_MODEL), jnp.float32),
        jax.ShapeDtypeStruct((BATCH,), jnp.float32),
    ],
    scratch_types=[
        pltpu.VMEM((NCH, CHUNK), jnp.int32),
        pltpu.VMEM((B_PER_W, D_MODEL), jnp.float32),
        pltpu.VMEM((B_PER_W,), jnp.float32),
    ]
    + [pltpu.SemaphoreType.DMA for _ in range(NCH)]
    + [pltpu.SemaphoreType.DMA, pltpu.SemaphoreType.DMA],
)
def _gather_kernel(idx_hbm, weight_hbm, bias_hbm, w_out, b_out,
                   idx_v, rows_v, bias_v, *sems):
    gsems, bsem, ssem = sems[:NCH], sems[NCH], sems[NCH + 1]
    wid = lax.axis_index("s") * NC + lax.axis_index("c")
    base = wid * B_PER_W
    pltpu.sync_copy(idx_hbm.at[wid], idx_v)
    wcopies = []
    for j in range(NCH):
        wcopies.append(pltpu.async_copy(
            weight_hbm.at[idx_v.at[j]],
            rows_v.at[pl.ds(j * CHUNK, CHUNK)], gsems[j]))
    bcopies = []
    for j in range(NCH):
        bcopies.append(pltpu.async_copy(
            bias_hbm.at[idx_v.at[j]],
            bias_v.at[pl.ds(j * CHUNK, CHUNK)], bsem))
    stores = []
    for j in range(NCH):
        wcopies[j].wait()
        stores.append(pltpu.async_copy(
            rows_v.at[pl.ds(j * CHUNK, CHUNK)],
            w_out.at[pl.ds(base + j * CHUNK, CHUNK)], ssem))
    for c in bcopies:
        c.wait()
    stores.append(pltpu.async_copy(bias_v, b_out.at[pl.ds(base, B_PER_W)], ssem))
    for s in stores:
        s.wait()


def kernel(coords, voxel_indices, weight, bias):
    del coords
    idx = voxel_indices.astype(jnp.int32).reshape(NW, NCH, CHUNK)
    w, b = _gather_kernel(idx, weight, bias)
    return (w, b)
